# baseline (speedup 1.0000x reference)
import functools

import jax
import jax.numpy as jnp
from jax import lax
from jax.experimental import pallas as pl
from jax.experimental.pallas import tpu as pltpu
from jax.experimental.pallas import tpu_sc as plsc

N = 10000
E = 320000
D = 128
C = 64

NC = 2
NS = 16
NW = NC * NS
CHUNK = 128
NPAD = 10240
ROWS_PER_TILE = NPAD // NS

PH = 4
CPT = 78
STEPS = 9
TAIL = CPT - STEPS * 2 * PH
ECHUNKS = E // CHUNK
NUM_HOPS = 4


def _mm_body(x_ref, w_ref, o_ref):
    o_ref[...] = lax.dot_general(
        x_ref[...], w_ref[...], (((1,), (1,)), ((), ())),
        preferred_element_type=jnp.float32)


def _project(featp, W):
    return pl.pallas_call(
        _mm_body,
        grid=(5,),
        in_specs=[
            pl.BlockSpec((2048, D), lambda i: (i, 0)),
            pl.BlockSpec((C, D), lambda i: (0, 0)),
        ],
        out_specs=pl.BlockSpec((2048, C), lambda i: (i, 0)),
        out_shape=jax.ShapeDtypeStruct((NPAD, C), jnp.float32),
    )(featp, W)


_MESH = plsc.VectorSubcoreMesh(core_axis_name="c", subcore_axis_name="s")

RPW = NPAD // NW


@functools.partial(
    pl.kernel,
    out_type=jax.ShapeDtypeStruct((NPAD, C), jnp.float32),
    mesh=_MESH,
    scratch_types=[
        pltpu.VMEM((RPW, C), jnp.float32),
        pltpu.VMEM((RPW, C), jnp.float32),
        pltpu.VMEM((C,), jnp.float32),
        pltpu.SemaphoreType.DMA,
    ],
    compiler_params=pltpu.CompilerParams(use_tc_tiling_on_sc=False),
)
def _combine(p_hbm, bias_hbm, y_hbm, va, vb, vbias, sem):
    cid = lax.axis_index("c")
    sid = lax.axis_index("s")
    r0 = (sid * NC + cid) * RPW
    c1 = pltpu.async_copy(p_hbm.at[0, pl.ds(r0, RPW)], va, sem)
    c2 = pltpu.async_copy(p_hbm.at[1, pl.ds(r0, RPW)], vb, sem)
    c3 = pltpu.async_copy(bias_hbm, vbias, sem)
    c1.wait()
    c2.wait()
    c3.wait()
    bv = [vbias[pl.ds(k * 16, 16)] for k in range(C // 16)]

    def row(r, carry):
        for k in range(C // 16):
            cs = pl.ds(k * 16, 16)
            va[r, cs] = va[r, cs] + vb[r, cs] + bv[k]
        return carry

    lax.fori_loop(0, RPW, row, 0)
    pltpu.sync_copy(va, y_hbm.at[pl.ds(r0, RPW)])


@functools.partial(
    pl.kernel,
    out_type=jax.ShapeDtypeStruct((NC, NPAD, C), jnp.float32),
    mesh=_MESH,
    scratch_types=[
        pltpu.VMEM((CPT + 1, CHUNK), jnp.int32),
        pltpu.VMEM((CPT + 1, CHUNK), jnp.int32),
        pltpu.VMEM((2 * PH, CHUNK, C), jnp.float32),
        pltpu.VMEM_SHARED((NPAD, C), jnp.float32),
        pltpu.SemaphoreType.DMA,
        pltpu.SemaphoreType.DMA,
        pltpu.SemaphoreType.DMA,
        pltpu.SemaphoreType.DMA,
        pltpu.SemaphoreType.DMA,
    ],
    compiler_params=pltpu.CompilerParams(use_tc_tiling_on_sc=False),
)
def _hop(src_hbm, dst_hbm, x_hbm, zeros_hbm, out_hbm,
         src_v, dst_v, rows_v, accum, gsem0, gsem1, ssem0, ssem1, isem):
    cid = lax.axis_index("c")
    sid = lax.axis_index("s")
    wid = sid * NC + cid
    r0 = sid * ROWS_PER_TILE

    base = CPT * wid + jnp.minimum(wid, 4)
    ci = pltpu.async_copy(src_hbm.at[pl.ds(base, CPT + 1)], src_v, isem)
    cj = pltpu.async_copy(dst_hbm.at[pl.ds(base, CPT + 1)], dst_v, isem)
    cz = pltpu.async_copy(zeros_hbm.at[pl.ds(r0, ROWS_PER_TILE)],
                          accum.at[pl.ds(r0, ROWS_PER_TILE)], isem)
    ci.wait()
    cj.wait()
    cz.wait()
    plsc.subcore_barrier()

    def gather(chunk, slot, sem):
        return pltpu.async_copy(
            x_hbm.at[src_v.at[chunk]], rows_v.at[slot], sem)

    def scatter(chunk, slot, sem):
        return pltpu.async_copy(
            rows_v.at[slot], accum.at[dst_v.at[chunk]], sem, add=True)

    def drain(slot, sem):
        pltpu.make_async_copy(
            x_hbm.at[pl.ds(0, CHUNK)], rows_v.at[slot], sem).wait()

    for b in range(PH):
        gather(b, b, gsem0)

    def body(t, carry):
        e0 = 2 * t * PH
        for b in range(PH):
            drain(b, gsem0)
        sc_e = [scatter(e0 + b, b, ssem0) for b in range(PH)]

        @pl.when(t > 0)
        def _():
            for b in range(PH):
                drain(PH + b, ssem1)

        g_o = [gather(e0 + PH + b, PH + b, gsem1) for b in range(PH)]
        for dsc in g_o:
            dsc.wait()
        [scatter(e0 + PH + b, PH + b, ssem1) for b in range(PH)]
        for dsc in sc_e:
            dsc.wait()

        @pl.when(t + 1 < STEPS)
        def _():
            for b in range(PH):
                gather(e0 + 2 * PH + b, b, gsem0)
        return carry

    lax.fori_loop(0, STEPS, body, 0)
    for b in range(PH):
        drain(PH + b, ssem1)

    gs = [gather(STEPS * 2 * PH + b, b, gsem0) for b in range(TAIL)]
    ss = []
    for b in range(TAIL):
        gs[b].wait()
        ss.append(scatter(STEPS * 2 * PH + b, b, ssem0))
    for dsc in ss:
        dsc.wait()

    @pl.when(wid < 4)
    def _():
        gather(CPT, TAIL, gsem0).wait()
        scatter(CPT, TAIL, ssem0).wait()

    plsc.subcore_barrier()
    pltpu.sync_copy(accum.at[pl.ds(r0, ROWS_PER_TILE)],
                    out_hbm.at[cid, pl.ds(r0, ROWS_PER_TILE)])


def kernel(feat, edge_index, W, b):
    featp = jnp.pad(feat, ((0, NPAD - N), (0, 0)))
    X = _project(featp, W)

    dst2 = jnp.pad(edge_index[0], (0, CHUNK)).reshape(ECHUNKS + 1, CHUNK)
    src2 = jnp.pad(edge_index[1], (0, CHUNK)).reshape(ECHUNKS + 1, CHUNK)

    zeros = jnp.zeros((NPAD, C), jnp.float32)
    zero_bias = jnp.zeros((C,), jnp.float32)
    Y = X
    for i in range(NUM_HOPS):
        P = _hop(src2, dst2, Y, zeros)
        Y = _combine(P, b if i == NUM_HOPS - 1 else zero_bias)
    return Y[:N]
